# baseline (speedup 1.0000x reference)
import functools

import jax
import jax.numpy as jnp
from jax import lax
from jax.experimental import pallas as pl
from jax.experimental.pallas import tpu as pltpu
from jax.experimental.pallas import tpu_sc as plsc

_S, _B, _D = 2048, 4, 1024
_L = 16
_NC, _NS = 2, 16
_NW = _NC * _NS
_RPW = _S // _NW
_R = 8
_NCH = _RPW // _R
_BD = _B * _D


def _sc_body(x_hbm, pos_hbm, out_hbm, xbuf, pbuf, sx0, sx1, sp0, sp1, so0, so1):
    wid = lax.axis_index("s") * _NC + lax.axis_index("c")
    base = wid * _RPW
    sx = (sx0, sx1)
    sp = (sp0, sp1)
    so = (so0, so1)
    loads = [None, None]
    stores = [None, None]

    def start_load(g):
        b = g % 2
        row0 = base + g * _R
        cx = pltpu.async_copy(x_hbm.at[pl.ds(row0, _R)], xbuf.at[b], sx[b])
        cp = pltpu.async_copy(pos_hbm.at[pl.ds(row0, _R)], pbuf.at[b], sp[b])
        loads[b] = (cx, cp)

    start_load(0)
    for g in range(_NCH):
        b = g % 2
        if g + 1 < _NCH:
            if stores[1 - b] is not None:
                stores[1 - b].wait()
            start_load(g + 1)
        cx, cp = loads[b]
        cx.wait()
        cp.wait()
        for s in range(_R):
            def jbody(j, carry, _b=b, _s=s):
                off = j * _L
                pvec = pbuf[_b, _s, pl.ds(off, _L)]
                for bb in range(_B):
                    plsc.addupdate(xbuf.at[_b, _s, pl.ds(bb * _D + off, _L)], pvec)
                return carry
            lax.fori_loop(0, _D // _L, jbody, 0)
        stores[b] = pltpu.async_copy(
            xbuf.at[b], out_hbm.at[pl.ds(base + g * _R, _R)], so[b])
    stores[0].wait()
    stores[1].wait()


@functools.partial(jax.jit, static_argnames=())
def _sc_add(x2d, pos_table):
    run = pl.kernel(
        _sc_body,
        out_type=jax.ShapeDtypeStruct((_S, _BD), jnp.float32),
        mesh=plsc.VectorSubcoreMesh(
            core_axis_name="c", subcore_axis_name="s",
            num_cores=_NC, num_subcores=_NS),
        scratch_types=[
            pltpu.VMEM((2, _R, _BD), jnp.float32),
            pltpu.VMEM((2, _R, _D), jnp.float32),
            pltpu.SemaphoreType.DMA,
            pltpu.SemaphoreType.DMA,
            pltpu.SemaphoreType.DMA,
            pltpu.SemaphoreType.DMA,
            pltpu.SemaphoreType.DMA,
            pltpu.SemaphoreType.DMA,
        ],
    )
    return run(x2d, pos_table)


def kernel(x, pos_table):
    S, B, D = x.shape
    out = _sc_add(x.reshape(S, B * D), pos_table[:S])
    return out.reshape(S, B, D)
